# baseline (device time: 36192 ns/iter reference)
import jax
import jax.numpy as jnp
from jax import lax
from jax.experimental import pallas as pl
from jax.experimental.pallas import tpu as pltpu

N_DEV = 4
B_SH = 64
B = N_DEV * B_SH
HALF = B // 2
D = 512
H_SH = 1024
N_RDMA = 14

F32 = jnp.float32
BF16 = jnp.bfloat16


def kernel(x, Win0, Wout0, Win1, Wout1, Win2, Wout2):
    def body(x_ref, win0_ref, wout0_ref, win1_ref, wout1_ref, win2_ref,
             wout2_ref, out_ref, xfull_ref, part_ref, xb_ref, agb_ref,
             sab_ref, arb_ref, rsb_s_ref, rsb_r_ref, winb_ref, woutb_ref,
             send_sems, recv_sems):
        my = lax.axis_index("i")
        y_p = my ^ 1
        x_p = 3 - my
        d_p = (3 - my) ^ 1

        barrier_sem = pltpu.get_barrier_semaphore()
        for nbr in (y_p, x_p):
            pl.semaphore_signal(
                barrier_sem, inc=1,
                device_id=(nbr,), device_id_type=pl.DeviceIdType.MESH,
            )
        pl.semaphore_wait(barrier_sem, 2)

        sem_counter = [0]

        def rdma(src, dst, target):
            i = sem_counter[0]
            sem_counter[0] += 1
            return pltpu.make_async_remote_copy(
                src_ref=src, dst_ref=dst,
                send_sem=send_sems.at[i], recv_sem=recv_sems.at[i],
                device_id=(target,), device_id_type=pl.DeviceIdType.MESH,
            )

        def rows(c):
            return pl.ds(c * B_SH, B_SH)

        def load_weights(slot, win_ref, wout_ref):
            winb_ref[slot] = win_ref[:, :].astype(BF16)
            woutb_ref[slot] = wout_ref[:, :].astype(BF16)

        def mlp_chunk(v_bf16, slot):
            h = jnp.dot(v_bf16, winb_ref[slot], preferred_element_type=F32)
            hb = jnp.maximum(h, 0.0).astype(BF16)
            return jnp.dot(hb, woutb_ref[slot], preferred_element_type=F32)

        ds_a = pl.ds(0, HALF)
        ds_b = pl.ds(HALF, HALF)

        xb_ref[:, :] = x_ref[:, :].astype(BF16)
        r0 = rdma(xb_ref, agb_ref.at[0], y_p)
        r1 = rdma(xb_ref, agb_ref.at[1], x_p)
        r2 = rdma(xb_ref, agb_ref.at[2], d_p)
        r0.start()
        r1.start()
        r2.start()
        load_weights(0, win0_ref, wout0_ref)
        part_ref[rows(my), :] = mlp_chunk(xb_ref[:, :], 0)
        r0.wait_recv()
        part_ref[rows(y_p), :] = mlp_chunk(agb_ref[0], 0)
        r1.wait_recv()
        part_ref[rows(x_p), :] = mlp_chunk(agb_ref[1], 0)
        r2.wait_recv()
        part_ref[rows(d_p), :] = mlp_chunk(agb_ref[2], 0)

        sab_ref[0] = part_ref[ds_a, :].astype(BF16)
        ra = rdma(sab_ref.at[0], arb_ref.at[0], y_p)
        ra.start()
        sab_ref[1] = part_ref[ds_b, :].astype(BF16)
        rb = rdma(sab_ref.at[1], arb_ref.at[1], x_p)
        rb.start()
        load_weights(1, win1_ref, wout1_ref)
        ra.wait_recv()
        xfull_ref[ds_a, :] = part_ref[ds_a, :] + arb_ref[0].astype(F32)
        sab_ref[2] = xfull_ref[ds_a, :].astype(BF16)
        ra2 = rdma(sab_ref.at[2], arb_ref.at[2], x_p)
        ra2.start()
        rb.wait_recv()
        xfull_ref[ds_b, :] = part_ref[ds_b, :] + arb_ref[1].astype(F32)
        sab_ref[3] = xfull_ref[ds_b, :].astype(BF16)
        rb2 = rdma(sab_ref.at[3], arb_ref.at[3], y_p)
        rb2.start()
        ra2.wait_recv()
        xfull_ref[ds_a, :] = xfull_ref[ds_a, :] + arb_ref[2].astype(F32)
        part_ref[ds_a, :] = mlp_chunk(xfull_ref[ds_a, :].astype(BF16), 1)
        sab_ref[4] = part_ref[ds_a, :].astype(BF16)
        sa = rdma(sab_ref.at[4], arb_ref.at[4], y_p)
        sa.start()
        rb2.wait_recv()
        xfull_ref[ds_b, :] = xfull_ref[ds_b, :] + arb_ref[3].astype(F32)
        part_ref[ds_b, :] = mlp_chunk(xfull_ref[ds_b, :].astype(BF16), 1)
        sab_ref[5] = part_ref[ds_b, :].astype(BF16)
        sb = rdma(sab_ref.at[5], arb_ref.at[5], x_p)
        sb.start()
        load_weights(2, win2_ref, wout2_ref)
        sa.wait_recv()
        xfull_ref[ds_a, :] = part_ref[ds_a, :] + arb_ref[4].astype(F32)
        sab_ref[6] = xfull_ref[ds_a, :].astype(BF16)
        sa2 = rdma(sab_ref.at[6], arb_ref.at[6], x_p)
        sa2.start()
        sb.wait_recv()
        xfull_ref[ds_b, :] = part_ref[ds_b, :] + arb_ref[5].astype(F32)
        sab_ref[7] = xfull_ref[ds_b, :].astype(BF16)
        sb2 = rdma(sab_ref.at[7], arb_ref.at[7], y_p)
        sb2.start()
        sa2.wait_recv()
        xfull_ref[ds_a, :] = xfull_ref[ds_a, :] + arb_ref[6].astype(F32)
        part_ref[ds_a, :] = mlp_chunk(xfull_ref[ds_a, :].astype(BF16), 2)
        sb2.wait_recv()
        xfull_ref[ds_b, :] = xfull_ref[ds_b, :] + arb_ref[7].astype(F32)
        part_ref[ds_b, :] = mlp_chunk(xfull_ref[ds_b, :].astype(BF16), 2)

        rsb_s_ref[0] = part_ref[rows(y_p), :].astype(BF16)
        rq0 = rdma(rsb_s_ref.at[0], rsb_r_ref.at[0], y_p)
        rq0.start()
        rsb_s_ref[1] = part_ref[rows(x_p), :].astype(BF16)
        rq1 = rdma(rsb_s_ref.at[1], rsb_r_ref.at[1], x_p)
        rq1.start()
        rsb_s_ref[2] = part_ref[rows(d_p), :].astype(BF16)
        rq2 = rdma(rsb_s_ref.at[2], rsb_r_ref.at[2], d_p)
        rq2.start()
        rq0.wait_recv()
        rq1.wait_recv()
        rq2.wait_recv()
        out_ref[:, :] = (part_ref[rows(my), :]
                         + rsb_r_ref[0].astype(F32)
                         + rsb_r_ref[1].astype(F32)
                         + rsb_r_ref[2].astype(F32))

        for r in (r0, r1, r2, ra, rb, ra2, rb2, sa, sb, sa2, sb2,
                  rq0, rq1, rq2):
            r.wait_send()

    return pl.pallas_call(
        body,
        out_shape=jax.ShapeDtypeStruct((B_SH, D), F32),
        in_specs=[pl.BlockSpec(memory_space=pltpu.VMEM)] * 7,
        out_specs=pl.BlockSpec(memory_space=pltpu.VMEM),
        scratch_shapes=[
            pltpu.VMEM((B, D), F32),
            pltpu.VMEM((B, D), F32),
            pltpu.VMEM((B_SH, D), BF16),
            pltpu.VMEM((3, B_SH, D), BF16),
            pltpu.VMEM((8, HALF, D), BF16),
            pltpu.VMEM((8, HALF, D), BF16),
            pltpu.VMEM((3, B_SH, D), BF16),
            pltpu.VMEM((3, B_SH, D), BF16),
            pltpu.VMEM((3, D, H_SH), BF16),
            pltpu.VMEM((3, H_SH, D), BF16),
            pltpu.SemaphoreType.DMA((N_RDMA,)),
            pltpu.SemaphoreType.DMA((N_RDMA,)),
        ],
        compiler_params=pltpu.CompilerParams(collective_id=0),
    )(x, Win0, Wout0, Win1, Wout1, Win2, Wout2)


# device time: 35008 ns/iter; 1.0338x vs baseline; 1.0338x over previous
import jax
import jax.numpy as jnp
from jax import lax
from jax.experimental import pallas as pl
from jax.experimental.pallas import tpu as pltpu

N_DEV = 4
B_SH = 64
B = N_DEV * B_SH
HALF = B // 2
D = 512
H_SH = 1024
N_RDMA = 14

F32 = jnp.float32
BF16 = jnp.bfloat16


def kernel(x, Win0, Wout0, Win1, Wout1, Win2, Wout2):
    def body(x_ref, win0_ref, wout0_ref, win1_ref, wout1_ref, win2_ref,
             wout2_ref, out_ref, partb_ref, xb_ref, agb_ref, sab_ref,
             arb_ref, rsb_r_ref, winb_ref, woutb_ref, send_sems, recv_sems):
        my = lax.axis_index("i")
        y_p = my ^ 1
        x_p = 3 - my
        d_p = (3 - my) ^ 1

        barrier_sem = pltpu.get_barrier_semaphore()
        for nbr in (y_p, x_p):
            pl.semaphore_signal(
                barrier_sem, inc=1,
                device_id=(nbr,), device_id_type=pl.DeviceIdType.MESH,
            )
        pl.semaphore_wait(barrier_sem, 2)

        sem_counter = [0]

        def rdma(src, dst, target):
            i = sem_counter[0]
            sem_counter[0] += 1
            return pltpu.make_async_remote_copy(
                src_ref=src, dst_ref=dst,
                send_sem=send_sems.at[i], recv_sem=recv_sems.at[i],
                device_id=(target,), device_id_type=pl.DeviceIdType.MESH,
            )

        def rows(c):
            return pl.ds(c * B_SH, B_SH)

        def load_weights(slot, win_ref, wout_ref):
            winb_ref[slot] = win_ref[:, :].astype(BF16)
            woutb_ref[slot] = wout_ref[:, :].astype(BF16)

        def mlp_chunk(v_bf16, slot):
            h = jnp.dot(v_bf16, winb_ref[slot], preferred_element_type=F32)
            hb = jnp.maximum(h, 0.0).astype(BF16)
            return jnp.dot(hb, woutb_ref[slot],
                           preferred_element_type=F32).astype(BF16)

        ds_a = pl.ds(0, HALF)
        ds_b = pl.ds(HALF, HALF)

        xb_ref[:, :] = x_ref[:, :].astype(BF16)
        r0 = rdma(xb_ref, agb_ref.at[0], y_p)
        r1 = rdma(xb_ref, agb_ref.at[1], x_p)
        r2 = rdma(xb_ref, agb_ref.at[2], d_p)
        r0.start()
        r1.start()
        r2.start()
        load_weights(0, win0_ref, wout0_ref)
        partb_ref[rows(my), :] = mlp_chunk(xb_ref[:, :], 0)
        r0.wait_recv()
        partb_ref[rows(y_p), :] = mlp_chunk(agb_ref[0], 0)
        r1.wait_recv()
        partb_ref[rows(x_p), :] = mlp_chunk(agb_ref[1], 0)
        r2.wait_recv()
        partb_ref[rows(d_p), :] = mlp_chunk(agb_ref[2], 0)

        ra = rdma(partb_ref.at[ds_a], arb_ref.at[0], y_p)
        ra.start()
        rb = rdma(partb_ref.at[ds_b], arb_ref.at[1], x_p)
        rb.start()
        load_weights(1, win1_ref, wout1_ref)
        ra.wait_recv()
        sab_ref[0] = partb_ref[ds_a, :] + arb_ref[0]
        ra2 = rdma(sab_ref.at[0], arb_ref.at[2], x_p)
        ra2.start()
        rb.wait_recv()
        sab_ref[1] = partb_ref[ds_b, :] + arb_ref[1]
        rb2 = rdma(sab_ref.at[1], arb_ref.at[3], y_p)
        rb2.start()
        ra2.wait_recv()
        ra.wait_send()
        partb_ref[ds_a, :] = mlp_chunk(sab_ref[0] + arb_ref[2], 1)
        sa = rdma(partb_ref.at[ds_a], arb_ref.at[4], y_p)
        sa.start()
        rb2.wait_recv()
        rb.wait_send()
        partb_ref[ds_b, :] = mlp_chunk(sab_ref[1] + arb_ref[3], 1)
        sb = rdma(partb_ref.at[ds_b], arb_ref.at[5], x_p)
        sb.start()
        load_weights(2, win2_ref, wout2_ref)
        sa.wait_recv()
        sab_ref[2] = partb_ref[ds_a, :] + arb_ref[4]
        sa2 = rdma(sab_ref.at[2], arb_ref.at[6], x_p)
        sa2.start()
        sb.wait_recv()
        sab_ref[3] = partb_ref[ds_b, :] + arb_ref[5]
        sb2 = rdma(sab_ref.at[3], arb_ref.at[7], y_p)
        sb2.start()
        sa2.wait_recv()
        sa.wait_send()
        partb_ref[ds_a, :] = mlp_chunk(sab_ref[2] + arb_ref[6], 2)
        sb2.wait_recv()
        sb.wait_send()
        partb_ref[ds_b, :] = mlp_chunk(sab_ref[3] + arb_ref[7], 2)

        rq0 = rdma(partb_ref.at[rows(y_p)], rsb_r_ref.at[0], y_p)
        rq0.start()
        rq1 = rdma(partb_ref.at[rows(x_p)], rsb_r_ref.at[1], x_p)
        rq1.start()
        rq2 = rdma(partb_ref.at[rows(d_p)], rsb_r_ref.at[2], d_p)
        rq2.start()
        rq0.wait_recv()
        rq1.wait_recv()
        rq2.wait_recv()
        out_ref[:, :] = (partb_ref[rows(my), :].astype(F32)
                         + rsb_r_ref[0].astype(F32)
                         + rsb_r_ref[1].astype(F32)
                         + rsb_r_ref[2].astype(F32))

        for r in (r0, r1, r2, ra2, rb2, sa2, sb2, rq0, rq1, rq2):
            r.wait_send()

    return pl.pallas_call(
        body,
        out_shape=jax.ShapeDtypeStruct((B_SH, D), F32),
        in_specs=[pl.BlockSpec(memory_space=pltpu.VMEM)] * 7,
        out_specs=pl.BlockSpec(memory_space=pltpu.VMEM),
        scratch_shapes=[
            pltpu.VMEM((B, D), BF16),
            pltpu.VMEM((B_SH, D), BF16),
            pltpu.VMEM((3, B_SH, D), BF16),
            pltpu.VMEM((4, HALF, D), BF16),
            pltpu.VMEM((8, HALF, D), BF16),
            pltpu.VMEM((3, B_SH, D), BF16),
            pltpu.VMEM((3, D, H_SH), BF16),
            pltpu.VMEM((3, H_SH, D), BF16),
            pltpu.SemaphoreType.DMA((N_RDMA,)),
            pltpu.SemaphoreType.DMA((N_RDMA,)),
        ],
        compiler_params=pltpu.CompilerParams(collective_id=0),
    )(x, Win0, Wout0, Win1, Wout1, Win2, Wout2)
